# baseline (speedup 1.0000x reference)
import functools

import jax
import jax.numpy as jnp
from jax import lax
from jax.experimental import pallas as pl
from jax.experimental.pallas import tpu as pltpu
from jax.experimental.pallas import tpu_sc as plsc

BATCH = 16384
HIDDEN = 64

_info = plsc.get_sparse_core_info()
_NC, _NS, _L = _info.num_cores, _info.num_subcores, _info.num_lanes
_NW = _NC * _NS
_BPW = BATCH // _NW

_mesh = plsc.VectorSubcoreMesh(core_axis_name="c", subcore_axis_name="s")


@functools.partial(
    pl.kernel,
    mesh=_mesh,
    out_type=jax.ShapeDtypeStruct((BATCH, HIDDEN), jnp.float32),
    scratch_types=[
        pltpu.VMEM((_BPW,), jnp.int32),
        pltpu.VMEM((_BPW,), jnp.float32),
        pltpu.VMEM((_BPW, HIDDEN), jnp.float32),
        pltpu.SemaphoreType.DMA,
    ],
    compiler_params=pltpu.CompilerParams(use_tc_tiling_on_sc=False),
)
def _scale_gather(time_hbm, word_hbm, table_hbm, out_hbm,
                  idx_v, time_v, rows_v, sem):
    wid = lax.axis_index("s") * _NC + lax.axis_index("c")
    base = wid * _BPW
    pltpu.sync_copy(word_hbm.at[pl.ds(base, _BPW)], idx_v)
    pltpu.sync_copy(time_hbm.at[pl.ds(base, _BPW)], time_v)
    pltpu.async_copy(table_hbm.at[idx_v], rows_v, sem).wait()

    def body(g, _):
        tvec = time_v[pl.ds(g * _L, _L)]
        for r2 in range(_L):
            t = jnp.full((_L,), tvec[r2])
            r = g * _L + r2
            for j in range(HIDDEN // _L):
                sl = pl.ds(j * _L, _L)
                rows_v[r, sl] = rows_v[r, sl] * t
        return ()

    lax.fori_loop(0, _BPW // _L, body, ())
    pltpu.sync_copy(rows_v, out_hbm.at[pl.ds(base, _BPW)])


def kernel(_time, word, para_embedding):
    return _scale_gather(_time, word.astype(jnp.int32), para_embedding)


# speedup vs baseline: 2.5641x; 2.5641x over previous
import functools

import jax
import jax.numpy as jnp
from jax import lax
from jax.experimental import pallas as pl
from jax.experimental.pallas import tpu as pltpu
from jax.experimental.pallas import tpu_sc as plsc

BATCH = 16384
VOCAB = 1000000
HIDDEN = 64
_GRP = 8

_info = plsc.get_sparse_core_info()
_NC, _NS, _L = _info.num_cores, _info.num_subcores, _info.num_lanes
_NW = _NC * _NS
_BPW = BATCH // _NW

_mesh = plsc.VectorSubcoreMesh(core_axis_name="c", subcore_axis_name="s")


@functools.partial(
    pl.kernel,
    mesh=_mesh,
    out_type=jax.ShapeDtypeStruct((BATCH, HIDDEN), jnp.float32),
    scratch_types=[
        pltpu.VMEM((_BPW,), jnp.int32),
        pltpu.VMEM((_BPW,), jnp.float32),
        pltpu.VMEM((_BPW, HIDDEN), jnp.float32),
        pltpu.SemaphoreType.DMA,
    ],
)
def _scale_gather(time_hbm, word_hbm, table3_hbm, out_hbm,
                  widx_v, time_v, rows_v, sem):
    wid = lax.axis_index("s") * _NC + lax.axis_index("c")
    base = wid * _BPW
    pltpu.sync_copy(word_hbm.at[pl.ds(base, _BPW)], widx_v)
    pltpu.sync_copy(time_hbm.at[pl.ds(base, _BPW)], time_v)

    def issue_body(g, _):
        wv = widx_v[pl.ds(g * _L, _L)]
        bv = jnp.right_shift(wv, 3)
        sv = jnp.bitwise_and(wv, _GRP - 1)
        for r2 in range(_L):
            pltpu.async_copy(
                table3_hbm.at[bv[r2], sv[r2]],
                rows_v.at[g * _L + r2],
                sem,
            )
        return ()

    lax.fori_loop(0, _BPW // _L, issue_body, ())
    pltpu.make_async_copy(out_hbm.at[pl.ds(base, _BPW)], rows_v, sem).wait()

    def scale_body(g, _):
        tvec = time_v[pl.ds(g * _L, _L)]
        for r2 in range(_L):
            t = jnp.full((_L,), tvec[r2])
            r = g * _L + r2
            for j in range(HIDDEN // _L):
                sl = pl.ds(j * _L, _L)
                rows_v[r, sl] = rows_v[r, sl] * t
        return ()

    lax.fori_loop(0, _BPW // _L, scale_body, ())
    pltpu.sync_copy(rows_v, out_hbm.at[pl.ds(base, _BPW)])


def kernel(_time, word, para_embedding):
    table3 = jnp.reshape(para_embedding, (VOCAB // _GRP, _GRP, HIDDEN))
    return _scale_gather(_time, word.astype(jnp.int32), table3)
